# baseline (speedup 1.0000x reference)
import functools

import jax
import jax.numpy as jnp
from jax import lax
from jax.experimental import pallas as pl
from jax.experimental.pallas import tpu as pltpu
from jax.experimental.pallas import tpu_sc as plsc

N = 32768
E = 524288
BS = 256
F_IN = 128
HID = 512
CNN = 256
E2 = E + N
NC, NS = 2, 16
NW = NC * NS
FCH = 16
ER = E2 // 128
ER_W = ER // NW
ER_S = ER // NS
BROWS = 8
NB = ER_S // BROWS

_mesh = lambda: plsc.VectorSubcoreMesh(
    core_axis_name="c", subcore_axis_name="s", num_cores=NC, num_subcores=NS)


def _deg_body(colm, ewm, out, colbuf, ewbuf, degpart):
    c = lax.axis_index("c")
    s = lax.axis_index("s")
    w = s * NC + c
    zf = jnp.zeros((16,), jnp.float32)

    def zero(i, _):
        degpart[pl.ds(i * 16, 16)] = zf
        return 0
    lax.fori_loop(0, N // 16, zero, 0)

    base = w * ER_W
    pltpu.sync_copy(colm.at[pl.ds(base, ER_W)], colbuf)
    pltpu.sync_copy(ewm.at[pl.ds(base, ER_W)], ewbuf)

    def body(r, _):
        for k in range(8):
            idx = colbuf[r, pl.ds(k * 16, 16)]
            wv = ewbuf[r, pl.ds(k * 16, 16)]
            plsc.addupdate_scatter(degpart, [idx], wv)
        return 0
    lax.fori_loop(0, ER_W, body, 0)
    pltpu.sync_copy(degpart, out.at[w])


def _deg_call(colm, ewm):
    return pl.kernel(
        _deg_body,
        out_type=jax.ShapeDtypeStruct((NW, N), jnp.float32),
        mesh=_mesh(),
        scratch_types=[
            pltpu.VMEM((ER_W, 128), jnp.int32),
            pltpu.VMEM((ER_W, 128), jnp.float32),
            pltpu.VMEM((N,), jnp.float32),
        ],
        compiler_params=pltpu.CompilerParams(needs_layout_passes=False),
    )(colm, ewm)


def _scale_body(parts_ref, x_ref, dx_ref, d2_ref):
    sdeg = jnp.sum(parts_ref[...], axis=0)
    dv = jnp.where(sdeg > 0, lax.rsqrt(sdeg), 0.0)
    dx_ref[...] = x_ref[...] * dv[:, None]
    d2_ref[...] = dv.reshape(-1, 1)


def _scale_call(parts, x):
    return pl.pallas_call(
        _scale_body,
        grid=(16,),
        in_specs=[
            pl.BlockSpec((NW, N // 16), lambda i: (0, i)),
            pl.BlockSpec((N // 16, F_IN), lambda i: (i, 0)),
        ],
        out_specs=(
            pl.BlockSpec((N // 16, F_IN), lambda i: (i, 0)),
            pl.BlockSpec((N // 16, 1), lambda i: (i, 0)),
        ),
        out_shape=(
            jax.ShapeDtypeStruct((N, F_IN), jnp.float32),
            jax.ShapeDtypeStruct((N, 1), jnp.float32),
        ),
    )(parts, x)


def _prop_body(x8, rowm, colm, ewm, out_h,
               rowF0, rowF1, colF, ewF0, ewF1, gidx0, gidx1, rows0, rows1,
               zbuf, accum, semg0, semg1, sems0, sems1):
    c = lax.axis_index("c")
    s = lax.axis_index("s")
    zf = jnp.zeros((16,), jnp.float32)

    def zero(i, _):
        zbuf[i, pl.ds(0, 16)] = zf
        return 0
    lax.fori_loop(0, 256, zero, 0)

    pltpu.sync_copy(colm.at[pl.ds(s * ER_S, ER_S)], colF)

    def mk(gidx, rows, semg):
        return [pltpu.make_async_copy(
            x8.at[gidx.at[k]], rows.at[pl.ds(k * 128, 128)], semg.at[k])
            for k in range(BROWS)]

    def prep(rowF, gidx, gds, coff):
        for k in range(BROWS):
            def gx(j, _):
                gidx[k, pl.ds(j * 16, 16)] = (
                    rowF[k, pl.ds(j * 16, 16)] + coff)
                return 0
            lax.fori_loop(0, 8, gx, 0)
            gds[k].start()

    def process(gds, rows, ewF, rl, sem_s):
        for k in range(BROWS):
            gds[k].wait()

            def scale(j, _):
                ew16 = ewF[k, pl.ds(j * 16, 16)]
                for u in range(16):
                    nv = ew16[u]
                    row = k * 128 + j * 16 + u
                    rows[row, pl.ds(0, 16)] = rows[row, pl.ds(0, 16)] * nv
                return 0
            lax.fori_loop(0, 8, scale, 0)
            pltpu.make_async_copy(
                rows.at[pl.ds(k * 128, 128)], accum.at[colF.at[rl + k]],
                sem_s).start(add=True)

    def drain(rows, sem_s):
        for k in range(BROWS):
            pltpu.make_async_copy(
                rows.at[pl.ds(k * 128, 128)], accum.at[colF.at[k]],
                sem_s).wait()

    def chunk_body(cl, _):
        coff = (c * 4 + cl) * N
        for z in range(8):
            pltpu.sync_copy(zbuf, accum.at[pl.ds(s * 2048 + z * 256, 256)])
        plsc.subcore_barrier()
        gds0 = mk(gidx0, rows0, semg0)
        gds1 = mk(gidx1, rows1, semg1)
        pltpu.sync_copy(rowm.at[pl.ds(s * ER_S, BROWS)], rowF0)
        pltpu.sync_copy(ewm.at[pl.ds(s * ER_S, BROWS)], ewF0)
        prep(rowF0, gidx0, gds0, coff)

        def tbody(t, _):
            b0 = 2 * t
            b1 = 2 * t + 1
            pltpu.sync_copy(
                rowm.at[pl.ds(s * ER_S + b1 * BROWS, BROWS)], rowF1)
            pltpu.sync_copy(
                ewm.at[pl.ds(s * ER_S + b1 * BROWS, BROWS)], ewF1)

            @pl.when(t > 0)
            def _():
                drain(rows1, sems1)
            prep(rowF1, gidx1, gds1, coff)
            process(gds0, rows0, ewF0, b0 * BROWS, sems0)

            @pl.when(t < NB // 2 - 1)
            def _():
                pltpu.sync_copy(
                    rowm.at[pl.ds(s * ER_S + (b0 + 2) * BROWS, BROWS)],
                    rowF0)
                pltpu.sync_copy(
                    ewm.at[pl.ds(s * ER_S + (b0 + 2) * BROWS, BROWS)],
                    ewF0)
            drain(rows0, sems0)

            @pl.when(t < NB // 2 - 1)
            def _():
                prep(rowF0, gidx0, gds0, coff)
            process(gds1, rows1, ewF1, b1 * BROWS, sems1)
            return 0
        lax.fori_loop(0, NB // 2, tbody, 0)
        drain(rows1, sems1)
        plsc.subcore_barrier()
        pltpu.sync_copy(accum.at[pl.ds(s * 2048, 2048)],
                        out_h.at[pl.ds(coff + s * 2048, 2048)])
        plsc.subcore_barrier()
        return 0
    lax.fori_loop(0, 4, chunk_body, 0)


def _prop_call(x8, rowm, colm, ewm):
    return pl.kernel(
        _prop_body,
        out_type=jax.ShapeDtypeStruct((8 * N, FCH), jnp.float32),
        mesh=_mesh(),
        scratch_types=[
            pltpu.VMEM((BROWS, 128), jnp.int32),
            pltpu.VMEM((BROWS, 128), jnp.int32),
            pltpu.VMEM((ER_S, 128), jnp.int32),
            pltpu.VMEM((BROWS, 128), jnp.float32),
            pltpu.VMEM((BROWS, 128), jnp.float32),
            pltpu.VMEM((BROWS, 128), jnp.int32),
            pltpu.VMEM((BROWS, 128), jnp.int32),
            pltpu.VMEM((1024, FCH), jnp.float32),
            pltpu.VMEM((1024, FCH), jnp.float32),
            pltpu.VMEM((256, FCH), jnp.float32),
            pltpu.VMEM_SHARED((N, FCH), jnp.float32),
            pltpu.SemaphoreType.DMA((BROWS,)),
            pltpu.SemaphoreType.DMA((BROWS,)),
            pltpu.SemaphoreType.DMA,
            pltpu.SemaphoreType.DMA,
        ],
        compiler_params=pltpu.CompilerParams(
            needs_layout_passes=False, use_tc_tiling_on_sc=False),
    )(x8, rowm, colm, ewm)


def _head_body(p8_ref, d2_ref, w1, w2, b2r, wc1, bc1r, wc2, bc2r, wl2r, blr,
               o_ref):
    p8 = p8_ref[...]
    d2 = d2_ref[...]
    w12 = jnp.dot(w1[...], w2[...], preferred_element_type=jnp.float32)
    acc12 = jnp.broadcast_to(b2r[...], (2048, HID)).astype(jnp.float32)
    for gi in range(8):
        acc12 = acc12 + jnp.dot(
            p8[gi] * d2, w12[gi * FCH:(gi + 1) * FCH, :],
            preferred_element_type=jnp.float32)
    g = jnp.tanh(acc12)
    g4 = g.reshape(16, 4, 32, HID)
    acc = jnp.broadcast_to(bc1r[...], (512, HID)).astype(jnp.float32)
    for gi in range(4):
        ggi = g4[:, gi].reshape(512, HID)
        wslice = wc1[...][:, gi * HID:(gi + 1) * HID]
        acc = acc + lax.dot_general(
            ggi, wslice, (((1,), (1,)), ((), ())),
            preferred_element_type=jnp.float32)
    a = jax.nn.relu(acc)
    bm = jax.nn.relu(lax.dot_general(
        a, wc2[...], (((1,), (1,)), ((), ())),
        preferred_element_type=jnp.float32) + bc2r[...])
    bm3 = bm.reshape(16, 32, CNN)
    pr = bm3 * wl2r[...][None, :, :]
    sv = jnp.sum(jnp.sum(pr, axis=2), axis=1) + blr[0, 0]
    o_ref[...] = jax.nn.sigmoid(sv).reshape(1, 1, 16)


def _head_call(p, d2, W1, W2, b2, Wc1, bc1, Wc2, bc2, wl2, bl):
    full = lambda shape: pl.BlockSpec(shape, lambda i: tuple(0 for _ in shape))
    return pl.pallas_call(
        _head_body,
        grid=(16,),
        in_specs=[
            pl.BlockSpec((8, 2048, FCH), lambda i: (0, i, 0)),
            pl.BlockSpec((2048, 1), lambda i: (i, 0)),
            full((F_IN, F_IN)), full((F_IN, HID)), full((1, HID)),
            full((HID, 4 * HID)), full((1, HID)),
            full((CNN, HID)), full((1, CNN)),
            full((32, CNN)), full((1, 1)),
        ],
        out_specs=pl.BlockSpec((1, 1, 16), lambda i: (i, 0, 0)),
        out_shape=jax.ShapeDtypeStruct((16, 1, 16), jnp.float32),
    )(p, d2, W1, W2, b2, Wc1, bc1, Wc2, bc2, wl2, bl)


def kernel(x, edge_index, edge_attr, batch_vec, W1, W2, b2, Wc1, bc1, Wc2,
           bc2, Wl, bl):
    loop = jnp.arange(N, dtype=jnp.int32)
    rowm = jnp.concatenate([edge_index[0], loop]).reshape(ER, 128)
    colm = jnp.concatenate([edge_index[1], loop]).reshape(ER, 128)
    ewm = jnp.concatenate(
        [edge_attr, jnp.ones((N,), jnp.float32)]).reshape(ER, 128)
    parts = _deg_call(colm, ewm)
    dinvx, dinv2 = _scale_call(parts, x)
    x8 = dinvx.reshape(N, 8, FCH).transpose(1, 0, 2).reshape(8 * N, FCH)
    prop8 = _prop_call(x8, rowm, colm, ewm).reshape(8, N, FCH)
    out = _head_call(prop8, dinv2, W1, W2, b2.reshape(1, HID), Wc1,
                     bc1.reshape(1, HID), Wc2, bc2.reshape(1, CNN),
                     Wl.reshape(CNN, 32).T, bl.reshape(1, 1))
    return out.reshape(-1)


# speedup vs baseline: 1.0401x; 1.0401x over previous
import functools

import jax
import jax.numpy as jnp
from jax import lax
from jax.experimental import pallas as pl
from jax.experimental.pallas import tpu as pltpu
from jax.experimental.pallas import tpu_sc as plsc

N = 32768
E = 524288
BS = 256
F_IN = 128
HID = 512
CNN = 256
E2 = E + N
NC, NS = 2, 16
NW = NC * NS
FCH = 16
ER = E2 // 128
ER_W = ER // NW
ER_S = ER // NS
BROWS = 8
NB = ER_S // BROWS

_mesh = lambda: plsc.VectorSubcoreMesh(
    core_axis_name="c", subcore_axis_name="s", num_cores=NC, num_subcores=NS)


def _deg_body(colm, ewm, out, colbuf, ewbuf, degpart):
    c = lax.axis_index("c")
    s = lax.axis_index("s")
    w = s * NC + c
    zf = jnp.zeros((16,), jnp.float32)

    def zero(i, _):
        degpart[pl.ds(i * 16, 16)] = zf
        return 0
    lax.fori_loop(0, N // 16, zero, 0)

    base = w * ER_W
    pltpu.sync_copy(colm.at[pl.ds(base, ER_W)], colbuf)
    pltpu.sync_copy(ewm.at[pl.ds(base, ER_W)], ewbuf)

    def body(r, _):
        for k in range(8):
            idx = colbuf[r, pl.ds(k * 16, 16)]
            wv = ewbuf[r, pl.ds(k * 16, 16)]
            plsc.addupdate_scatter(degpart, [idx], wv)
        return 0
    lax.fori_loop(0, ER_W, body, 0)
    pltpu.sync_copy(degpart, out.at[w])


def _deg_call(colm, ewm):
    return pl.kernel(
        _deg_body,
        out_type=jax.ShapeDtypeStruct((NW, N), jnp.float32),
        mesh=_mesh(),
        scratch_types=[
            pltpu.VMEM((ER_W, 128), jnp.int32),
            pltpu.VMEM((ER_W, 128), jnp.float32),
            pltpu.VMEM((N,), jnp.float32),
        ],
        compiler_params=pltpu.CompilerParams(needs_layout_passes=False),
    )(colm, ewm)


def _scale_body(parts_ref, x_ref, dx_ref, d2_ref):
    sdeg = jnp.sum(parts_ref[...], axis=0)
    dv = jnp.where(sdeg > 0, lax.rsqrt(sdeg), 0.0)
    dx_ref[...] = x_ref[...] * dv[:, None]
    d2_ref[...] = dv.reshape(-1, 1)


def _scale_call(parts, x):
    return pl.pallas_call(
        _scale_body,
        grid=(16,),
        in_specs=[
            pl.BlockSpec((NW, N // 16), lambda i: (0, i)),
            pl.BlockSpec((N // 16, F_IN), lambda i: (i, 0)),
        ],
        out_specs=(
            pl.BlockSpec((N // 16, F_IN), lambda i: (i, 0)),
            pl.BlockSpec((N // 16, 1), lambda i: (i, 0)),
        ),
        out_shape=(
            jax.ShapeDtypeStruct((N, F_IN), jnp.float32),
            jax.ShapeDtypeStruct((N, 1), jnp.float32),
        ),
    )(parts, x)


def _prop_body(x8, rowm, colm, ewm, out_h,
               rowF0, rowF1, colF, ewF0, ewF1, gidx0, gidx1, rows0, rows1,
               zbuf, accum, semg0, semg1, sems0, sems1):
    c = lax.axis_index("c")
    s = lax.axis_index("s")
    zf = jnp.zeros((16,), jnp.float32)

    def zero(i, _):
        zbuf[i, pl.ds(0, 16)] = zf
        return 0
    lax.fori_loop(0, 256, zero, 0)

    pltpu.sync_copy(colm.at[pl.ds(s * ER_S, ER_S)], colF)

    def mk(gidx, rows, semg):
        return [pltpu.make_async_copy(
            x8.at[gidx.at[k]], rows.at[pl.ds(k * 128, 128)], semg.at[k])
            for k in range(BROWS)]

    def prep(rowF, gidx, gds, coff):
        for k in range(BROWS):
            def gx(j, _):
                gidx[k, pl.ds(j * 16, 16)] = (
                    rowF[k, pl.ds(j * 16, 16)] + coff)
                return 0
            lax.fori_loop(0, 8, gx, 0)
            gds[k].start()

    def process(gds, rows, ewF, rl, sem_s):
        for k in range(BROWS):
            gds[k].wait()

            def scale(j, _):
                ew16 = ewF[k, pl.ds(j * 16, 16)]
                for u in range(16):
                    nv = ew16[u]
                    row = k * 128 + j * 16 + u
                    rows[row, pl.ds(0, 16)] = rows[row, pl.ds(0, 16)] * nv
                return 0
            lax.fori_loop(0, 8, scale, 0)
            pltpu.make_async_copy(
                rows.at[pl.ds(k * 128, 128)], accum.at[colF.at[rl + k]],
                sem_s).start(add=True)

    def drain(rows, sem_s):
        for k in range(BROWS):
            pltpu.make_async_copy(
                rows.at[pl.ds(k * 128, 128)], accum.at[colF.at[k]],
                sem_s).wait()

    def chunk_body(cl, _):
        coff = (c * 4 + cl) * N
        for z in range(8):
            pltpu.sync_copy(zbuf, accum.at[pl.ds(s * 2048 + z * 256, 256)])
        plsc.subcore_barrier()
        gds0 = mk(gidx0, rows0, semg0)
        gds1 = mk(gidx1, rows1, semg1)
        pltpu.sync_copy(rowm.at[pl.ds(s * ER_S, BROWS)], rowF0)
        pltpu.sync_copy(ewm.at[pl.ds(s * ER_S, BROWS)], ewF0)
        prep(rowF0, gidx0, gds0, coff)

        def tbody(t, _):
            b0 = 2 * t
            b1 = 2 * t + 1
            pltpu.sync_copy(
                rowm.at[pl.ds(s * ER_S + b1 * BROWS, BROWS)], rowF1)
            pltpu.sync_copy(
                ewm.at[pl.ds(s * ER_S + b1 * BROWS, BROWS)], ewF1)

            @pl.when(t > 0)
            def _():
                drain(rows1, sems1)
            prep(rowF1, gidx1, gds1, coff)
            process(gds0, rows0, ewF0, b0 * BROWS, sems0)

            @pl.when(t < NB // 2 - 1)
            def _():
                pltpu.sync_copy(
                    rowm.at[pl.ds(s * ER_S + (b0 + 2) * BROWS, BROWS)],
                    rowF0)
                pltpu.sync_copy(
                    ewm.at[pl.ds(s * ER_S + (b0 + 2) * BROWS, BROWS)],
                    ewF0)
            drain(rows0, sems0)

            @pl.when(t < NB // 2 - 1)
            def _():
                prep(rowF0, gidx0, gds0, coff)
            process(gds1, rows1, ewF1, b1 * BROWS, sems1)
            return 0
        lax.fori_loop(0, NB // 2, tbody, 0)
        drain(rows1, sems1)
        plsc.subcore_barrier()
        pltpu.sync_copy(accum.at[pl.ds(s * 2048, 2048)],
                        out_h.at[pl.ds(coff + s * 2048, 2048)])
        plsc.subcore_barrier()
        return 0
    lax.fori_loop(0, 4, chunk_body, 0)


def _prop_call(x8, rowm, colm, ewm):
    return pl.kernel(
        _prop_body,
        out_type=jax.ShapeDtypeStruct((8 * N, FCH), jnp.float32),
        mesh=_mesh(),
        scratch_types=[
            pltpu.VMEM((BROWS, 128), jnp.int32),
            pltpu.VMEM((BROWS, 128), jnp.int32),
            pltpu.VMEM((ER_S, 128), jnp.int32),
            pltpu.VMEM((BROWS, 128), jnp.float32),
            pltpu.VMEM((BROWS, 128), jnp.float32),
            pltpu.VMEM((BROWS, 128), jnp.int32),
            pltpu.VMEM((BROWS, 128), jnp.int32),
            pltpu.VMEM((1024, FCH), jnp.float32),
            pltpu.VMEM((1024, FCH), jnp.float32),
            pltpu.VMEM((256, FCH), jnp.float32),
            pltpu.VMEM_SHARED((N, FCH), jnp.float32),
            pltpu.SemaphoreType.DMA((BROWS,)),
            pltpu.SemaphoreType.DMA((BROWS,)),
            pltpu.SemaphoreType.DMA,
            pltpu.SemaphoreType.DMA,
        ],
        compiler_params=pltpu.CompilerParams(
            needs_layout_passes=False, use_tc_tiling_on_sc=False),
    )(x8, rowm, colm, ewm)


def _head_body(p_ref, d2_ref, w1, w2, b2r, wc1, bc1r, wc2, bc2r, wl2r, blr,
               o_ref):
    p = p_ref[...] * d2_ref[...]
    w12 = jnp.dot(w1[...], w2[...], preferred_element_type=jnp.float32)
    g = jnp.tanh(jnp.dot(p, w12, preferred_element_type=jnp.float32)
                 + b2r[...])
    g4 = g.reshape(16, 4, 32, HID)
    acc = jnp.broadcast_to(bc1r[...], (512, HID)).astype(jnp.float32)
    for gi in range(4):
        ggi = g4[:, gi].reshape(512, HID)
        wslice = wc1[...][:, gi * HID:(gi + 1) * HID]
        acc = acc + lax.dot_general(
            ggi, wslice, (((1,), (1,)), ((), ())),
            preferred_element_type=jnp.float32)
    a = jax.nn.relu(acc)
    bm = jax.nn.relu(lax.dot_general(
        a, wc2[...], (((1,), (1,)), ((), ())),
        preferred_element_type=jnp.float32) + bc2r[...])
    bm3 = bm.reshape(16, 32, CNN)
    pr = bm3 * wl2r[...][None, :, :]
    sv = jnp.sum(jnp.sum(pr, axis=2), axis=1) + blr[0, 0]
    o_ref[...] = jax.nn.sigmoid(sv).reshape(1, 1, 16)


def _head_call(p, d2, W1, W2, b2, Wc1, bc1, Wc2, bc2, wl2, bl):
    full = lambda shape: pl.BlockSpec(shape, lambda i: tuple(0 for _ in shape))
    return pl.pallas_call(
        _head_body,
        grid=(16,),
        in_specs=[
            pl.BlockSpec((2048, F_IN), lambda i: (i, 0)),
            pl.BlockSpec((2048, 1), lambda i: (i, 0)),
            full((F_IN, F_IN)), full((F_IN, HID)), full((1, HID)),
            full((HID, 4 * HID)), full((1, HID)),
            full((CNN, HID)), full((1, CNN)),
            full((32, CNN)), full((1, 1)),
        ],
        out_specs=pl.BlockSpec((1, 1, 16), lambda i: (i, 0, 0)),
        out_shape=jax.ShapeDtypeStruct((16, 1, 16), jnp.float32),
    )(p, d2, W1, W2, b2, Wc1, bc1, Wc2, bc2, wl2, bl)


def kernel(x, edge_index, edge_attr, batch_vec, W1, W2, b2, Wc1, bc1, Wc2,
           bc2, Wl, bl):
    loop = jnp.arange(N, dtype=jnp.int32)
    rowm = jnp.concatenate([edge_index[0], loop]).reshape(ER, 128)
    colm = jnp.concatenate([edge_index[1], loop]).reshape(ER, 128)
    ewm = jnp.concatenate(
        [edge_attr, jnp.ones((N,), jnp.float32)]).reshape(ER, 128)
    parts = _deg_call(colm, ewm)
    dinvx, dinv2 = _scale_call(parts, x)
    x8 = dinvx.reshape(N, 8, FCH).transpose(1, 0, 2).reshape(8 * N, FCH)
    prop8 = _prop_call(x8, rowm, colm, ewm)
    prop = prop8.reshape(8, N, FCH).transpose(1, 0, 2).reshape(N, F_IN)
    out = _head_call(prop, dinv2, W1, W2, b2.reshape(1, HID), Wc1,
                     bc1.reshape(1, HID), Wc2, bc2.reshape(1, CNN),
                     Wl.reshape(CNN, 32).T, bl.reshape(1, 1))
    return out.reshape(-1)
